# baseline (speedup 1.0000x reference)
import functools

import jax
import jax.numpy as jnp
from jax import lax
from jax.experimental import pallas as pl
from jax.experimental.pallas import tpu as pltpu
from jax.experimental.pallas import tpu_sc as plsc

BATCH = 4096
WIDTH = 26
DIM = 64
TOTAL = BATCH * WIDTH
CHUNK = 128
NBUF = 4

_info = plsc.get_sparse_core_info()
NC, NS = _info.num_cores, _info.num_subcores
NW = NC * NS
PER_W = TOTAL // NW
NSTEP = PER_W // CHUNK
assert PER_W * NW == TOTAL and NSTEP * CHUNK == PER_W


def _body(idx_hbm, table_hbm, out_hbm, idxs, rows, gsems, wsems):
    wid = lax.axis_index("s") * NC + lax.axis_index("c")
    base = wid * PER_W

    def zero_fixup(b, buf):
        def group(g, carry):
            iv = idxs[b][pl.ds(g * 16, 16)]
            m = iv == 0
            nz = jnp.max(plsc.all_reduce_population_count(m))

            @pl.when(nz > 0)
            def _():
                rid = g * 16 + jnp.arange(16, dtype=jnp.int32)
                zeros = jnp.zeros((16,), jnp.float32)

                def dcol(d, c):
                    cid = jnp.full((16,), d, jnp.int32)
                    plsc.store_scatter(buf, [rid, cid], zeros, mask=m)
                    return c

                lax.fori_loop(0, DIM, dcol, 0)

            return carry

        lax.fori_loop(0, CHUNK // 16, group, 0)

    def gather(s):
        b = s % NBUF
        pltpu.sync_copy(idx_hbm.at[pl.ds(base + s * CHUNK, CHUNK)], idxs[b])
        pltpu.async_copy(table_hbm.at[idxs[b]], rows[b], gsems[b])

    for s in range(NBUF):
        gather(s)

    for s in range(NSTEP):
        b = s % NBUF
        pltpu.make_async_copy(table_hbm.at[idxs[b]], rows[b], gsems[b]).wait()
        zero_fixup(b, rows[b])
        pltpu.sync_copy(rows[b], out_hbm.at[pl.ds(base + s * CHUNK, CHUNK)])
        if s + NBUF < NSTEP:
            gather(s + NBUF)


@functools.partial(jax.jit, static_argnums=())
def _lookup(idx_flat, table):
    mesh = plsc.VectorSubcoreMesh(core_axis_name="c", subcore_axis_name="s")
    scratch = [
        [pltpu.VMEM((CHUNK,), jnp.int32) for _ in range(NBUF)],
        [pltpu.VMEM((CHUNK, DIM), jnp.float32) for _ in range(NBUF)],
        [pltpu.SemaphoreType.DMA for _ in range(NBUF)],
        [pltpu.SemaphoreType.DMA for _ in range(NBUF)],
    ]
    k = pl.kernel(
        _body,
        mesh=mesh,
        out_type=jax.ShapeDtypeStruct((TOTAL, DIM), jnp.float32),
        scratch_types=scratch,
        compiler_params=pltpu.CompilerParams(
            use_tc_tiling_on_sc=False, needs_layout_passes=False
        ),
    )
    return k(idx_flat, table)


def kernel(input, table):
    idx_flat = input.astype(jnp.int32).reshape(TOTAL)
    out = _lookup(idx_flat, table)
    return out.reshape(BATCH, WIDTH, DIM)
